# baseline (speedup 1.0000x reference)
import functools

import jax
import jax.numpy as jnp
from jax import lax
from jax.experimental import pallas as pl
from jax.experimental.pallas import tpu as pltpu
from jax.experimental.pallas import tpu_sc as plsc

VOCAB = 1000
D = 64
B = 4096
S = 200

_info = plsc.get_sparse_core_info()
NC, NS = _info.num_cores, _info.num_subcores
NW = NC * NS
BPW = B // NW

K1 = 128
K2 = S - K1


def _emb_kernel(idx_hbm, table_hbm, out_hbm, idx_v, x0, x1, gs0, gs1, os0, os1):
    wid = lax.axis_index("s") * NC + lax.axis_index("c")
    b0 = wid * BPW
    pltpu.sync_copy(idx_hbm.at[pl.ds(b0, BPW)], idx_v)

    bufs = (x0, x1)
    gsems = (gs0, gs1)
    osems = (os0, os1)

    def do_row(i, p):
        d1 = pltpu.async_copy(
            table_hbm.at[idx_v.at[i, pl.ds(0, K1)]],
            bufs[p].at[pl.ds(0, K1)],
            gsems[p],
        )
        d2 = pltpu.async_copy(
            table_hbm.at[idx_v.at[i, pl.ds(K1, K2)]],
            bufs[p].at[pl.ds(K1, K2)],
            gsems[p],
        )
        d1.wait()
        d2.wait()
        pltpu.async_copy(bufs[p], out_hbm.at[b0 + i], osems[p])

    def reclaim(i, p):
        pltpu.make_async_copy(bufs[p], out_hbm.at[b0 + i], osems[p]).wait()

    do_row(0, 0)
    do_row(1, 1)

    def outer(t, carry):
        i0 = 2 + 2 * t
        for p in range(2):
            i = i0 + p
            reclaim(i - 2, p)
            do_row(i, p)
        return carry

    lax.fori_loop(0, (BPW - 2) // 2, outer, 0)

    for p in range(2):
        reclaim(BPW - 2 + p, p)


@jax.jit
def _emb(idx, table):
    mesh = plsc.VectorSubcoreMesh(core_axis_name="c", subcore_axis_name="s")
    run = functools.partial(
        pl.kernel,
        out_type=jax.ShapeDtypeStruct((B, S, D), jnp.float32),
        mesh=mesh,
        scratch_types=[
            pltpu.VMEM((BPW, S), jnp.int32),
            pltpu.VMEM((S, D), jnp.float32),
            pltpu.VMEM((S, D), jnp.float32),
            pltpu.SemaphoreType.DMA,
            pltpu.SemaphoreType.DMA,
            pltpu.SemaphoreType.DMA,
            pltpu.SemaphoreType.DMA,
        ],
        compiler_params=pltpu.CompilerParams(use_tc_tiling_on_sc=False),
    )(_emb_kernel)
    return run(idx, table)


def kernel(upos_encoded, embedding_weight):
    return _emb(upos_encoded.astype(jnp.int32), embedding_weight)


# speedup vs baseline: 1.1695x; 1.1695x over previous
import functools

import jax
import jax.numpy as jnp
from jax import lax
from jax.experimental import pallas as pl
from jax.experimental.pallas import tpu as pltpu
from jax.experimental.pallas import tpu_sc as plsc

VOCAB = 1000
D = 64
B = 4096
S = 200
TW = VOCAB * D

_info = plsc.get_sparse_core_info()
NC, NS = _info.num_cores, _info.num_subcores
NW = NC * NS
BPW = B // NW
LPW = BPW * S

CHUNK_ROWS = 32
CHUNK = CHUNK_ROWS * S
NCHUNK = BPW // CHUNK_ROWS


def _emb_kernel(idx_hbm, table_hbm, out_hbm, table_v, idx_v, x0, x1, os0, os1):
    wid = lax.axis_index("s") * NC + lax.axis_index("c")
    b0 = wid * BPW
    fbase = wid * LPW
    pltpu.sync_copy(table_hbm, table_v)

    bufs = (x0, x1)
    osems = (os0, os1)

    def chunk_body(c, carry):
        pltpu.sync_copy(idx_hbm.at[pl.ds(fbase + c * CHUNK, CHUNK)], idx_v.at[pl.ds(0, CHUNK)])

        def row_body(t, carry2):
            for p in range(2):
                lb = 2 * t + p
                bl = c * CHUNK_ROWS + lb

                @pl.when(bl >= 2)
                def _reclaim():
                    pltpu.make_async_copy(
                        bufs[p], out_hbm.at[b0 + bl - 2], osems[p]
                    ).wait()

                nb = lb * S

                def fill_body(sb, carry3):
                    rvec = idx_v[pl.ds(nb + sb * 8, 16)]
                    for u in range(8):
                        s = sb * 8 + u
                        base = rvec[u] * D
                        for k in range(4):
                            bufs[p][s, pl.ds(16 * k, 16)] = table_v[
                                pl.ds(base + 16 * k, 16)
                            ]
                    return carry3

                lax.fori_loop(0, S // 8, fill_body, 0)
                pltpu.async_copy(bufs[p], out_hbm.at[b0 + bl], osems[p])
            return carry2

        lax.fori_loop(0, CHUNK_ROWS // 2, row_body, 0)
        return carry

    lax.fori_loop(0, NCHUNK, chunk_body, 0)

    for p in range(2):
        pltpu.make_async_copy(
            bufs[p], out_hbm.at[b0 + BPW - 2 + p], osems[p]
        ).wait()


@jax.jit
def _emb(idx_flat, table_flat):
    mesh = plsc.VectorSubcoreMesh(core_axis_name="c", subcore_axis_name="s")
    run = functools.partial(
        pl.kernel,
        out_type=jax.ShapeDtypeStruct((B, S, D), jnp.float32),
        mesh=mesh,
        scratch_types=[
            pltpu.VMEM((TW,), jnp.float32),
            pltpu.VMEM((CHUNK + 16,), jnp.int32),
            pltpu.VMEM((S, D), jnp.float32),
            pltpu.VMEM((S, D), jnp.float32),
            pltpu.SemaphoreType.DMA,
            pltpu.SemaphoreType.DMA,
        ],
        compiler_params=pltpu.CompilerParams(use_tc_tiling_on_sc=True),
    )(_emb_kernel)
    return run(idx_flat, table_flat)


def kernel(upos_encoded, embedding_weight):
    idx_flat = upos_encoded.reshape(B * S).astype(jnp.int32)
    table_flat = embedding_weight.reshape(TW)
    return _emb(idx_flat, table_flat)
